# baseline (speedup 1.0000x reference)
import functools

import jax
import jax.numpy as jnp
from jax import lax
from jax.experimental import pallas as pl
from jax.experimental.pallas import tpu as pltpu
from jax.experimental.pallas import tpu_sc as plsc

VOCAB = 1_000_000
D = 64
P = 20
B = 1024
S = 200
OUT_S = P + S
NC = 2
NS = 16
NW = NC * NS
BPW = B // NW
C0 = 128
C1 = S - C0


def _gather_body(ids, table, prompt, out,
                 idx_lo0, idx_hi0, idx_lo1, idx_hi1,
                 ebuf0, ebuf1, gsem0, gsem1, osem0, osem1):
    wid = lax.axis_index("s") * NC + lax.axis_index("c")
    b0 = wid * BPW

    pltpu.sync_copy(prompt, ebuf0.at[pl.ds(0, P)])
    pltpu.sync_copy(prompt, ebuf1.at[pl.ds(0, P)])

    def step(i, carry):
        ba = b0 + 2 * i
        bb = ba + 1
        pltpu.sync_copy(ids.at[ba, pl.ds(0, C0)], idx_lo0)
        pltpu.sync_copy(ids.at[ba, pl.ds(C0, C1)], idx_hi0)
        ga1 = pltpu.async_copy(table.at[idx_lo0], ebuf0.at[pl.ds(P, C0)], gsem0)
        ga2 = pltpu.async_copy(table.at[idx_hi0], ebuf0.at[pl.ds(P + C0, C1)], gsem0)
        pltpu.sync_copy(ids.at[bb, pl.ds(0, C0)], idx_lo1)
        pltpu.sync_copy(ids.at[bb, pl.ds(C0, C1)], idx_hi1)
        gb1 = pltpu.async_copy(table.at[idx_lo1], ebuf1.at[pl.ds(P, C0)], gsem1)
        gb2 = pltpu.async_copy(table.at[idx_hi1], ebuf1.at[pl.ds(P + C0, C1)], gsem1)
        ga1.wait()
        ga2.wait()
        oa = pltpu.async_copy(ebuf0, out.at[ba], osem0)
        gb1.wait()
        gb2.wait()
        ob = pltpu.async_copy(ebuf1, out.at[bb], osem1)
        oa.wait()
        ob.wait()
        return carry

    lax.fori_loop(0, BPW // 2, step, 0)


_gather_call = functools.partial(
    pl.kernel,
    out_type=jax.ShapeDtypeStruct((B, OUT_S, D), jnp.float32),
    mesh=plsc.VectorSubcoreMesh(core_axis_name="c", subcore_axis_name="s"),
    compiler_params=pltpu.CompilerParams(use_tc_tiling_on_sc=False),
    scratch_types=[
        pltpu.VMEM((C0,), jnp.int32),
        pltpu.VMEM((C1,), jnp.int32),
        pltpu.VMEM((C0,), jnp.int32),
        pltpu.VMEM((C1,), jnp.int32),
        pltpu.VMEM((OUT_S, D), jnp.float32),
        pltpu.VMEM((OUT_S, D), jnp.float32),
        pltpu.SemaphoreType.DMA,
        pltpu.SemaphoreType.DMA,
        pltpu.SemaphoreType.DMA,
        pltpu.SemaphoreType.DMA,
    ],
)(_gather_body)


def _mask_body(am_ref, out_ref):
    out_ref[...] = jnp.concatenate(
        [jnp.ones((B, P), jnp.float32), am_ref[...]], axis=1)


def _mask_call(attention_mask):
    return pl.pallas_call(
        _mask_body,
        out_shape=jax.ShapeDtypeStruct((B, OUT_S), jnp.float32),
    )(attention_mask)


def kernel(input_ids, attention_mask, emb_table, prompt_table):
    ids = input_ids.astype(jnp.int32)
    emb_out = _gather_call(ids, emb_table, prompt_table)
    mask_out = _mask_call(attention_mask)
    return emb_out, mask_out


# speedup vs baseline: 1.5388x; 1.5388x over previous
import functools

import jax
import jax.numpy as jnp
from jax import lax
from jax.experimental import pallas as pl
from jax.experimental.pallas import tpu as pltpu
from jax.experimental.pallas import tpu_sc as plsc

VOCAB = 1_000_000
D = 64
P = 20
B = 1024
S = 200
OUT_S = P + S
NC = 2
NS = 16
NW = NC * NS
BPW = B // NW


def _gather_body(ids, table, prompt, out,
                 idx_all, obuf_a, obuf_b,
                 gsem_a, gsem_b, osem_a, osem_b):
    wid = lax.axis_index("s") * NC + lax.axis_index("c")
    b0 = wid * BPW

    pltpu.sync_copy(prompt, obuf_a.at[pl.ds(0, P)])
    pltpu.sync_copy(prompt, obuf_b.at[pl.ds(0, P)])

    bufs = ((obuf_a, gsem_a, osem_a), (obuf_b, gsem_b, osem_b))

    def pair(i2, carry):
        for u, (obuf, gsem, osem) in enumerate(bufs):
            b = b0 + 2 * i2 + u
            pltpu.sync_copy(ids.at[b], idx_all)

            @pl.when(2 * i2 + u >= 2)
            def _():
                pltpu.make_async_copy(
                    obuf.at[pl.ds(0, OUT_S)], out.at[b - 2], osem).wait()

            def group(g, carry2):
                v_vec = idx_all[pl.ds(16 * g, 16)]
                base = P + 16 * g
                for l in range(16):
                    pltpu.async_copy(table.at[v_vec[l]], obuf.at[base + l],
                                     gsem)
                return carry2

            lax.fori_loop(0, S // 16, group, 0)
            v_vec = idx_all[pl.ds(16 * (S // 16), 16)]
            for l in range(S - 16 * (S // 16)):
                pltpu.async_copy(table.at[v_vec[l]],
                                 obuf.at[P + 16 * (S // 16) + l], gsem)
            pltpu.make_async_copy(
                table.at[pl.ds(0, S)], obuf.at[pl.ds(P, S)], gsem).wait()
            pltpu.async_copy(obuf.at[pl.ds(0, OUT_S)], out.at[b], osem)
        return carry

    lax.fori_loop(0, BPW // 2, pair, 0)
    pltpu.make_async_copy(
        obuf_a.at[pl.ds(0, OUT_S)], out.at[b0 + BPW - 2], osem_a).wait()
    pltpu.make_async_copy(
        obuf_b.at[pl.ds(0, OUT_S)], out.at[b0 + BPW - 1], osem_b).wait()


_gather_call = functools.partial(
    pl.kernel,
    out_type=jax.ShapeDtypeStruct((B, OUT_S, D), jnp.float32),
    mesh=plsc.VectorSubcoreMesh(core_axis_name="c", subcore_axis_name="s"),
    scratch_types=[
        pltpu.VMEM((256,), jnp.int32),
        pltpu.VMEM((OUT_S, D), jnp.float32),
        pltpu.VMEM((OUT_S, D), jnp.float32),
        pltpu.SemaphoreType.DMA,
        pltpu.SemaphoreType.DMA,
        pltpu.SemaphoreType.DMA,
        pltpu.SemaphoreType.DMA,
    ],
)(_gather_body)


def _mask_body(am_ref, out_ref):
    out_ref[...] = jnp.concatenate(
        [jnp.ones((B, P), jnp.float32), am_ref[...]], axis=1)


def _mask_call(attention_mask):
    return pl.pallas_call(
        _mask_body,
        out_shape=jax.ShapeDtypeStruct((B, OUT_S), jnp.float32),
    )(attention_mask)


def kernel(input_ids, attention_mask, emb_table, prompt_table):
    ids = input_ids.astype(jnp.int32)
    ids = jnp.pad(ids, ((0, 0), (0, 256 - S)))
    emb_out = _gather_call(ids, emb_table, prompt_table)
    mask_out = _mask_call(attention_mask)
    return emb_out, mask_out
